# baseline (speedup 1.0000x reference)
import jax
import jax.numpy as jnp
from jax import lax
from jax.experimental import pallas as pl
from jax.experimental.pallas import tpu as pltpu
from jax.experimental.pallas import tpu_sc as plsc

MAXN = 1.0
NCORES = 2
NSUB = 16
NW = NCORES * NSUB
IW = 4096
KB8 = 8


def _rsqrt_nr(s):
    i = lax.bitcast_convert_type(s, jnp.int32)
    y = lax.bitcast_convert_type(jnp.int32(0x5F3759DF) - (i >> 1), jnp.float32)
    for _ in range(3):
        y = y * (1.5 - 0.5 * s * y * y)
    return y


def _lane_sum(v):
    idx = lax.iota(jnp.int32, 16)
    dnums = lax.GatherDimensionNumbers(
        offset_dims=(), collapsed_slice_dims=(0,), start_index_map=(0,)
    )
    for sh in (8, 4, 2, 1):
        perm = lax.gather(
            v, (idx ^ sh)[:, None], dnums, (1,),
            mode=lax.GatherScatterMode.PROMISE_IN_BOUNDS,
        )
        v = v + perm
    return v


def _sc_body(xt_hbm, w_hbm, y_hbm, w_v, spl0, spl1, xrow, out_v, sem_o):
    wid = lax.axis_index("s") * NCORES + lax.axis_index("c")
    ncols = xt_hbm.shape[0]
    nbatch = xt_hbm.shape[1]
    nih = IW // 16
    nchw = nbatch // IW

    pltpu.sync_copy(w_hbm, w_v)
    wn = []
    for t in range(2):
        qs = [w_v[t, pl.ds(16 * q, 16)] for q in range(4)]
        ss = qs[0] * qs[0] + qs[1] * qs[1] + qs[2] * qs[2] + qs[3] * qs[3]
        sv = _lane_sum(ss)
        normv = sv * _rsqrt_nr(sv)
        scalev = jnp.where(normv > MAXN, MAXN / (normv + 1e-7), 1.0)
        wn.append([qs[q] * scalev for q in range(4)])
    for k in range(64):
        q, ln = k // 16, k % 16
        spl0[k // 8, pl.ds(16 * (k % 8), 16)] = jnp.full((16,), wn[0][q][ln])
        spl1[k // 8, pl.ds(16 * (k % 8), 16)] = jnp.full((16,), wn[1][q][ln])

    nunits = (ncols * 64 // KB8) // NW

    @pl.loop(0, nunits)
    def _u(ul):
        u = nunits * wid + ul
        j = u // (64 // KB8)
        koct = u % (64 // KB8)

        @pl.when((ul == 0) | (koct == 0))
        def _():
            pltpu.sync_copy(xt_hbm.at[j], xrow)

        s0s = [spl0[koct, pl.ds(16 * kr, 16)] for kr in range(KB8)]
        s1s = [spl1[koct, pl.ds(16 * kr, 16)] for kr in range(KB8)]

        for ih in range(nchw):
            b = ih % 2
            ihoff = ih * IW

            if ih >= 2:
                pltpu.make_async_copy(
                    out_v.at[b],
                    y_hbm.at[0, pl.ds(0, KB8), pl.ds(0, IW)],
                    sem_o.at[b],
                ).wait()
            else:
                @pl.when(ul >= 1)
                def _():
                    pltpu.make_async_copy(
                        out_v.at[b],
                        y_hbm.at[0, pl.ds(0, KB8), pl.ds(0, IW)],
                        sem_o.at[b],
                    ).wait()

            @pl.loop(0, nih // 16)
            def _ib(ib):
                ms = [
                    xrow[pl.ds(ihoff + 16 * (16 * ib + u2), 16)] > 0
                    for u2 in range(16)
                ]
                for kr in range(KB8):
                    for u2 in range(16):
                        out_v[b, kr, pl.ds(16 * (16 * ib + u2), 16)] = (
                            jnp.where(ms[u2], s1s[kr], s0s[kr])
                        )

            pltpu.async_copy(
                out_v.at[b],
                y_hbm.at[j, pl.ds(KB8 * koct, KB8), pl.ds(ihoff, IW)],
                sem_o.at[b],
            )

    for b in range(2):
        pltpu.make_async_copy(
            out_v.at[b], y_hbm.at[0, pl.ds(0, KB8), pl.ds(0, IW)], sem_o.at[b]
        ).wait()


def kernel(x, weight):
    bsz, l = x.shape
    mesh = plsc.VectorSubcoreMesh(core_axis_name="c", subcore_axis_name="s")
    sc = pl.kernel(
        _sc_body,
        out_type=jax.ShapeDtypeStruct((l, 64, bsz), jnp.float32),
        mesh=mesh,
        compiler_params=pltpu.CompilerParams(use_tc_tiling_on_sc=True),
        scratch_types=[
            pltpu.VMEM((2, 64), jnp.float32),
            pltpu.VMEM((8, 128), jnp.float32),
            pltpu.VMEM((8, 128), jnp.float32),
            pltpu.VMEM((bsz,), jnp.int32),
            pltpu.VMEM((2, KB8, IW), jnp.float32),
            pltpu.SemaphoreType.DMA((2,)),
        ],
    )
    y = sc(x.T.astype(jnp.int32), weight)
    return y.transpose(2, 0, 1)


# speedup vs baseline: 1.4644x; 1.4644x over previous
import jax
import jax.numpy as jnp
from jax import lax
from jax.experimental import pallas as pl
from jax.experimental.pallas import tpu as pltpu
from jax.experimental.pallas import tpu_sc as plsc

MAXN = 1.0
NCORES = 2
NSUB = 16
NW = NCORES * NSUB
IW = 4096
KB8 = 8


def _rsqrt_nr(s):
    i = lax.bitcast_convert_type(s, jnp.int32)
    y = lax.bitcast_convert_type(jnp.int32(0x5F3759DF) - (i >> 1), jnp.float32)
    for _ in range(3):
        y = y * (1.5 - 0.5 * s * y * y)
    return y


def _lane_sum(v):
    idx = lax.iota(jnp.int32, 16)
    dnums = lax.GatherDimensionNumbers(
        offset_dims=(), collapsed_slice_dims=(0,), start_index_map=(0,)
    )
    for sh in (8, 4, 2, 1):
        perm = lax.gather(
            v, (idx ^ sh)[:, None], dnums, (1,),
            mode=lax.GatherScatterMode.PROMISE_IN_BOUNDS,
        )
        v = v + perm
    return v


def _sc_body(xt_hbm, w_hbm, y_hbm, w_v, spl0, spl1, xrow, out_v, sem_o):
    wid = lax.axis_index("s") * NCORES + lax.axis_index("c")
    ncols = xt_hbm.shape[0]
    nbatch = xt_hbm.shape[1]
    nih = IW // 16
    nchw = nbatch // IW

    pltpu.sync_copy(w_hbm, w_v)
    wn = []
    for t in range(2):
        qs = [w_v[t, pl.ds(16 * q, 16)] for q in range(4)]
        ss = qs[0] * qs[0] + qs[1] * qs[1] + qs[2] * qs[2] + qs[3] * qs[3]
        sv = _lane_sum(ss)
        normv = sv * _rsqrt_nr(sv)
        scalev = jnp.where(normv > MAXN, MAXN / (normv + 1e-7), 1.0)
        wn.append([qs[q] * scalev for q in range(4)])
    for k in range(64):
        q, ln = k // 16, k % 16
        spl0[k // 8, pl.ds(16 * (k % 8), 16)] = jnp.full((16,), wn[0][q][ln])
        spl1[k // 8, pl.ds(16 * (k % 8), 16)] = jnp.full((16,), wn[1][q][ln])

    nunits = (ncols * 64 // KB8) // NW

    @pl.loop(0, nunits)
    def _u(ul):
        u = nunits * wid + ul
        j = u // (64 // KB8)
        koct = u % (64 // KB8)

        @pl.when((ul == 0) | (koct == 0))
        def _():
            pltpu.sync_copy(xt_hbm.at[j], xrow)

        s0s = [spl0[koct, pl.ds(16 * kr, 16)] for kr in range(KB8)]
        s1s = [spl1[koct, pl.ds(16 * kr, 16)] for kr in range(KB8)]

        for ih in range(nchw):
            b = ih % 2
            ihoff = ih * IW

            if ih >= 2:
                pltpu.make_async_copy(
                    out_v.at[b],
                    y_hbm.at[0, pl.ds(0, KB8), pl.ds(0, IW)],
                    sem_o.at[b],
                ).wait()
            else:
                @pl.when(ul >= 1)
                def _():
                    pltpu.make_async_copy(
                        out_v.at[b],
                        y_hbm.at[0, pl.ds(0, KB8), pl.ds(0, IW)],
                        sem_o.at[b],
                    ).wait()

            @pl.loop(0, nih // 8, unroll=2)
            def _ib(ib):
                ms = [
                    xrow[pl.ds(ihoff + 16 * (8 * ib + u2), 16)] > 0
                    for u2 in range(8)
                ]
                for kr in range(KB8):
                    for u2 in range(8):
                        out_v[b, kr, pl.ds(16 * (8 * ib + u2), 16)] = (
                            jnp.where(ms[u2], s1s[kr], s0s[kr])
                        )

            pltpu.async_copy(
                out_v.at[b],
                y_hbm.at[j, pl.ds(KB8 * koct, KB8), pl.ds(ihoff, IW)],
                sem_o.at[b],
            )

    for b in range(2):
        pltpu.make_async_copy(
            out_v.at[b], y_hbm.at[0, pl.ds(0, KB8), pl.ds(0, IW)], sem_o.at[b]
        ).wait()


def kernel(x, weight):
    bsz, l = x.shape
    mesh = plsc.VectorSubcoreMesh(core_axis_name="c", subcore_axis_name="s")
    sc = pl.kernel(
        _sc_body,
        out_type=jax.ShapeDtypeStruct((l, 64, bsz), jnp.float32),
        mesh=mesh,
        compiler_params=pltpu.CompilerParams(use_tc_tiling_on_sc=True),
        scratch_types=[
            pltpu.VMEM((2, 64), jnp.float32),
            pltpu.VMEM((8, 128), jnp.float32),
            pltpu.VMEM((8, 128), jnp.float32),
            pltpu.VMEM((bsz,), jnp.int32),
            pltpu.VMEM((2, KB8, IW), jnp.float32),
            pltpu.SemaphoreType.DMA((2,)),
        ],
    )
    y = sc(x.T.astype(jnp.int32), weight)
    return y.transpose(2, 0, 1)
